# baseline (speedup 1.0000x reference)
import functools

import jax
import jax.numpy as jnp
from jax.experimental import pallas as pl
from jax.experimental.pallas import tpu as pltpu

TEMPERATURE = 1.0
EPS = 1e-20
B, V = 128, 100000

ROWS = 8
NROW = B // ROWS
FIX_W = 12544
NFIX = (V + FIX_W - 1) // FIX_W

INT_MAX = 2**31 - 1


def _argmax_zeros_kernel(l_ref, u_ref, o_ref, idx_ref):
    o_ref[...] = jnp.zeros_like(o_ref)
    g = -jnp.log(-jnp.log(u_ref[...] + EPS) + EPS)
    s = l_ref[...] + g
    bmax = jnp.max(s, axis=1, keepdims=True)
    col = jax.lax.broadcasted_iota(jnp.int32, s.shape, 1)
    idx_ref[...] = jnp.min(jnp.where(s == bmax, col, INT_MAX), axis=1,
                           keepdims=True)


def _row0_fixup_kernel(idx_ref, zeros_ref, o_ref):
    del zeros_ref
    j = pl.program_id(0)
    shape = o_ref.shape
    col = jax.lax.broadcasted_iota(jnp.int32, (B, shape[1]), 1) + j * shape[1]
    match = col == idx_ref[...]
    anyhot = jnp.any(match, axis=0, keepdims=True)
    row = jax.lax.broadcasted_iota(jnp.int32, shape, 0)
    o_ref[...] = jnp.where((row == 0) & anyhot, 1.0, 0.0).astype(jnp.float32)


@functools.partial(jax.jit, static_argnames=("interpret",))
def kernel(logits, gumbel_u, interpret=False):
    zeros, idx = pl.pallas_call(
        _argmax_zeros_kernel,
        grid=(NROW,),
        in_specs=[
            pl.BlockSpec((ROWS, V), lambda r: (r, 0)),
            pl.BlockSpec((ROWS, V), lambda r: (r, 0)),
        ],
        out_specs=[
            pl.BlockSpec((ROWS, V), lambda r: (r, 0)),
            pl.BlockSpec((ROWS, 1), lambda r: (r, 0)),
        ],
        out_shape=[
            jax.ShapeDtypeStruct((B, V), jnp.float32),
            jax.ShapeDtypeStruct((B, 1), jnp.int32),
        ],
        compiler_params=pltpu.CompilerParams(
            dimension_semantics=("parallel",)),
        interpret=interpret,
    )(logits, gumbel_u)

    out = pl.pallas_call(
        _row0_fixup_kernel,
        grid=(NFIX,),
        in_specs=[
            pl.BlockSpec((B, 1), lambda j: (0, 0)),
            pl.BlockSpec(memory_space=pl.ANY),
        ],
        out_specs=pl.BlockSpec((8, FIX_W), lambda j: (0, j)),
        out_shape=jax.ShapeDtypeStruct((B, V), jnp.float32),
        input_output_aliases={1: 0},
        compiler_params=pltpu.CompilerParams(
            dimension_semantics=("parallel",)),
        interpret=interpret,
    )(idx, zeros)
    return out


# speedup vs baseline: 1.0189x; 1.0189x over previous
import functools

import jax
import jax.numpy as jnp
from jax.experimental import pallas as pl
from jax.experimental.pallas import tpu as pltpu

TEMPERATURE = 1.0
EPS = 1e-20
B, V = 128, 100000

ROWS = 8
NROW = B // ROWS
VPAD = 100096
HOT_W = 4352
NHOT = VPAD // HOT_W

INT_MAX = 2**31 - 1


def _gumbel_argmax_kernel(l_ref, u_ref, hot_ref, idx_ref):
    r = pl.program_id(0)

    @pl.when(r < NROW)
    def _argmax():
        g = -jnp.log(-jnp.log(u_ref[...] + EPS) + EPS)
        s = l_ref[...] + g
        bmax = jnp.max(s, axis=1, keepdims=True)
        col = jax.lax.broadcasted_iota(jnp.int32, s.shape, 1)
        bidx = jnp.min(jnp.where(s == bmax, col, INT_MAX), axis=1,
                       keepdims=True)
        idx_ref[pl.ds(r * ROWS, ROWS), :] = bidx

    @pl.when(r == NROW)
    def _one_hot():
        idx = idx_ref[...]
        for c in range(NHOT):
            col = (jax.lax.broadcasted_iota(jnp.int32, (B, HOT_W), 1)
                   + c * HOT_W)
            anyhot = jnp.any(col == idx, axis=0, keepdims=True)
            hot_ref[:, pl.ds(c * HOT_W, HOT_W)] = anyhot.astype(jnp.float32)


@functools.partial(jax.jit, static_argnames=("interpret",))
def kernel(logits, gumbel_u, interpret=False):
    hot = pl.pallas_call(
        _gumbel_argmax_kernel,
        grid=(NROW + 1,),
        in_specs=[
            pl.BlockSpec((ROWS, V), lambda r: (jnp.minimum(r, NROW - 1), 0)),
            pl.BlockSpec((ROWS, V), lambda r: (jnp.minimum(r, NROW - 1), 0)),
        ],
        out_specs=pl.BlockSpec((1, VPAD), lambda r: (0, 0)),
        out_shape=jax.ShapeDtypeStruct((1, VPAD), jnp.float32),
        scratch_shapes=[pltpu.VMEM((B, 1), jnp.int32)],
        interpret=interpret,
    )(logits, gumbel_u)

    return jnp.concatenate(
        [hot[:, :V], jnp.zeros((B - 1, V), jnp.float32)], axis=0)


# speedup vs baseline: 1.0815x; 1.0615x over previous
import functools

import jax
import jax.numpy as jnp
from jax.experimental import pallas as pl
from jax.experimental.pallas import tpu as pltpu

TEMPERATURE = 1.0
EPS = 1e-20
B, V = 128, 100000

ROWS = 8
NROW = B // ROWS
VPAD = 100096
HOT_W = 4352
NHOT = VPAD // HOT_W

INT_MAX = 2**31 - 1


def _gumbel_argmax_kernel(l_ref, u_ref, hot_ref, idx_ref):
    r = pl.program_id(0)

    @pl.when(r < NROW)
    def _argmax():
        g = -jnp.log(-jnp.log(u_ref[...] + EPS) + EPS)
        s = l_ref[...] + g
        bmax = jnp.max(s, axis=1, keepdims=True)
        col = jax.lax.broadcasted_iota(jnp.int32, s.shape, 1)
        bidx = jnp.min(jnp.where(s == bmax, col, INT_MAX), axis=1,
                       keepdims=True)
        idx_ref[pl.ds(r * ROWS, ROWS), :] = bidx

    @pl.when(r == NROW)
    def _one_hot():
        idx = idx_ref[...]
        for c in range(NHOT):
            col = (jax.lax.broadcasted_iota(jnp.int32, (B, HOT_W), 1)
                   + c * HOT_W)
            anyhot = jnp.any(col == idx, axis=0, keepdims=True)
            hot_ref[:, pl.ds(c * HOT_W, HOT_W)] = anyhot.astype(jnp.float32)


@functools.partial(jax.jit, static_argnames=("interpret",))
def kernel(logits, gumbel_u, interpret=False):
    hot = pl.pallas_call(
        _gumbel_argmax_kernel,
        grid=(NROW + 1,),
        in_specs=[
            pl.BlockSpec((ROWS, V), lambda r: (jnp.minimum(r, NROW - 1), 0)),
            pl.BlockSpec((ROWS, V), lambda r: (jnp.minimum(r, NROW - 1), 0)),
        ],
        out_specs=pl.BlockSpec((1, VPAD), lambda r: (0, 0)),
        out_shape=jax.ShapeDtypeStruct((1, VPAD), jnp.float32),
        scratch_shapes=[pltpu.VMEM((B, 1), jnp.int32)],
        interpret=interpret,
    )(logits, gumbel_u)

    row_is_zero = jax.lax.broadcasted_iota(jnp.int32, (B, V), 0) == 0
    return jnp.where(row_is_zero, hot[:, :V], jnp.float32(0.0))
